# baseline (speedup 1.0000x reference)
import functools

import jax
import jax.numpy as jnp
from jax import lax
from jax.experimental import pallas as pl
from jax.experimental.pallas import tpu as pltpu
from jax.experimental.pallas import tpu_sc as plsc

CODEBOOK = 1024
DIM = 32
TBLK = 2048

_info = plsc.get_sparse_core_info()
_NW = _info.num_cores * _info.num_subcores


def _vq_block(z_ref, emb_ref, ids_ref, acc_ref):
    zb = z_ref[0]
    emb = emb_ref[...]
    emb_sq = jnp.sum(emb * emb, axis=1, keepdims=True)
    z_sq = jnp.sum(zb * zb, axis=0, keepdims=True)
    m2scores = jax.lax.dot_general(
        emb * jnp.float32(-2.0), zb, (((1,), (0,)), ((), ())),
        preferred_element_type=jnp.float32)
    d = (z_sq + emb_sq) + m2scores
    dmin = jnp.min(d, axis=0)
    iota = jax.lax.broadcasted_iota(jnp.int32, d.shape, 0).astype(jnp.float32)
    sel = jnp.where(d == dmin[None, :], iota, jnp.float32(CODEBOOK))
    ids_f = jnp.min(sel, axis=0)
    ids_ref[0, 0, :] = ids_f.astype(jnp.int32)

    part = jnp.sum(dmin).reshape(1, 1)
    step = pl.program_id(0) * pl.num_programs(1) + pl.program_id(1)

    @pl.when(step == 0)
    def _init():
        acc_ref[...] = part

    @pl.when(step != 0)
    def _accum():
        acc_ref[...] += part


def _sc_gather(table, idx, B, C, T):
    BT = idx.shape[0]
    b_per_w = BT // _NW
    w_per_b = T // b_per_w
    ngroups = b_per_w // 16
    mesh = plsc.VectorSubcoreMesh(core_axis_name="c", subcore_axis_name="s")

    @functools.partial(
        pl.kernel, mesh=mesh,
        out_type=jax.ShapeDtypeStruct((B, C, T), jnp.float32),
        compiler_params=pltpu.CompilerParams(needs_layout_passes=False),
        scratch_types=[
            pltpu.VMEM((CODEBOOK * DIM,), jnp.float32),
            pltpu.VMEM((b_per_w,), jnp.int32),
            pltpu.VMEM((DIM, b_per_w), jnp.float32),
        ],
    )
    def k(table_hbm, idx_hbm, out_hbm, table_v, idx_v, out_v):
        wid = lax.axis_index("s") * _info.num_cores + lax.axis_index("c")
        base = wid * b_per_w
        pltpu.sync_copy(table_hbm, table_v)
        pltpu.sync_copy(idx_hbm.at[pl.ds(base, b_per_w)], idx_v)

        def body(g, carry):
            flat = idx_v[pl.ds(g * 16, 16)] * DIM
            for dcol in range(DIM):
                out_v[dcol, pl.ds(g * 16, 16)] = plsc.load_gather(
                    table_v, [flat + dcol])
            return carry

        jax.lax.fori_loop(0, ngroups, body, 0)
        b_idx = wid // w_per_b
        t0 = (wid % w_per_b) * b_per_w
        pltpu.sync_copy(out_v, out_hbm.at[b_idx, :, pl.ds(t0, b_per_w)])

    return k(table.reshape(CODEBOOK * DIM), idx)


@jax.jit
def _vq(z, embedding_table):
    B, C, H, W = z.shape
    T = H * W
    nblk = T // TBLK
    z3 = z.reshape(B, C, T)

    grid = (B, nblk)
    ids3, acc = pl.pallas_call(
        _vq_block,
        grid=grid,
        in_specs=[
            pl.BlockSpec((1, C, TBLK), lambda b, t: (b, 0, t)),
            pl.BlockSpec((CODEBOOK, DIM), lambda b, t: (0, 0)),
        ],
        out_specs=[
            pl.BlockSpec((1, 1, TBLK), lambda b, t: (b * nblk + t, 0, 0)),
            pl.BlockSpec((1, 1), lambda b, t: (0, 0)),
        ],
        out_shape=[
            jax.ShapeDtypeStruct((B * nblk, 1, TBLK), jnp.int32),
            jax.ShapeDtypeStruct((1, 1), jnp.float32),
        ],
    )(z3, embedding_table)

    ids = ids3.reshape(B * T)
    zq3 = _sc_gather(embedding_table, ids, B, C, T)
    z_q = zq3.reshape(B, C, H, W)
    z_q = z + (z_q - z)
    mse = acc[0, 0] / (B * T * C)
    commitment_loss = 0.25 * mse
    codebook_loss = mse
    loss = commitment_loss + codebook_loss
    return (z_q, loss, commitment_loss, codebook_loss, ids)


def kernel(z, embedding_table):
    return _vq(z, embedding_table)


# speedup vs baseline: 1.7092x; 1.7092x over previous
import functools

import jax
import jax.numpy as jnp
from jax.experimental import pallas as pl
from jax.experimental.pallas import tpu as pltpu

CODEBOOK = 1024
DIM = 32
TBLK = 4096


def _vq_block(z_ref, emb_ref, zq_ref, ids_ref, acc_ref):
    zb = z_ref[0]
    emb = emb_ref[...]
    emb_sq = jnp.sum(emb * emb, axis=1, keepdims=True)
    z_sq = jnp.sum(zb * zb, axis=0, keepdims=True)
    m2scores = jax.lax.dot_general(
        emb * jnp.float32(-2.0), zb, (((1,), (0,)), ((), ())),
        preferred_element_type=jnp.float32)
    d = (z_sq + emb_sq) + m2scores
    dmin = jnp.min(d, axis=0)
    iota = jax.lax.broadcasted_iota(jnp.int32, d.shape, 0).astype(jnp.float32)
    sel = jnp.where(d == dmin[None, :], iota, jnp.float32(CODEBOOK))
    ids_f = jnp.min(sel, axis=0)
    ids_ref[0, 0, :] = ids_f.astype(jnp.int32)

    onehot = (iota == ids_f[None, :]).astype(jnp.float32)
    zq = jax.lax.dot_general(
        emb, onehot, (((0,), (0,)), ((), ())),
        preferred_element_type=jnp.float32)
    zq_ref[0] = zb + (zq - zb)

    part = jnp.sum(dmin).reshape(1, 1)
    step = pl.program_id(0) * pl.num_programs(1) + pl.program_id(1)

    @pl.when(step == 0)
    def _init():
        acc_ref[...] = part

    @pl.when(step != 0)
    def _accum():
        acc_ref[...] += part


@functools.partial(jax.jit, static_argnames=("interpret",))
def _vq(z, embedding_table, interpret=False):
    B, C, H, W = z.shape
    T = H * W
    nblk = T // TBLK
    z3 = z.reshape(B, C, T)

    grid = (B, nblk)
    zq3, ids3, acc = pl.pallas_call(
        _vq_block,
        grid=grid,
        in_specs=[
            pl.BlockSpec((1, C, TBLK), lambda b, t: (b, 0, t)),
            pl.BlockSpec((CODEBOOK, DIM), lambda b, t: (0, 0)),
        ],
        out_specs=[
            pl.BlockSpec((1, C, TBLK), lambda b, t: (b, 0, t)),
            pl.BlockSpec((1, 1, TBLK), lambda b, t: (b * nblk + t, 0, 0)),
            pl.BlockSpec((1, 1), lambda b, t: (0, 0)),
        ],
        out_shape=[
            jax.ShapeDtypeStruct((B, C, T), jnp.float32),
            jax.ShapeDtypeStruct((B * nblk, 1, TBLK), jnp.int32),
            jax.ShapeDtypeStruct((1, 1), jnp.float32),
        ],
        interpret=interpret,
    )(z3, embedding_table)

    z_q = zq3.reshape(B, C, H, W)
    ids = ids3.reshape(B * T)
    mse = acc[0, 0] / (B * T * C)
    commitment_loss = 0.25 * mse
    codebook_loss = mse
    loss = commitment_loss + codebook_loss
    return (z_q, loss, commitment_loss, codebook_loss, ids)


def kernel(z, embedding_table):
    return _vq(z, embedding_table)
